# Initial kernel scaffold; baseline (speedup 1.0000x reference)
import jax
import jax.numpy as jnp
from jax.experimental import pallas as pl


def kernel(sentence, word_vectors):
    raise NotImplementedError("write your pallas kernel here")


# speedup vs baseline: 6.5224x; 6.5224x over previous
import functools

import jax
import jax.numpy as jnp
from jax import lax
from jax.experimental import pallas as pl
from jax.experimental.pallas import tpu as pltpu
from jax.experimental.pallas import tpu_sc as plsc

_D = 64
_C = 128
_NW = 32


@functools.lru_cache(maxsize=None)
def _make_gather(n_tokens: int, vocab: int):
    assert n_tokens % (_NW * _C) == 0
    chunks_per_w = n_tokens // (_NW * _C)
    assert chunks_per_w % 2 == 0
    mesh = plsc.VectorSubcoreMesh(core_axis_name="c", subcore_axis_name="s")

    @functools.partial(
        pl.kernel,
        mesh=mesh,
        compiler_params=pltpu.CompilerParams(use_tc_tiling_on_sc=False),
        out_type=jax.ShapeDtypeStruct((n_tokens, _D), jnp.float32),
        scratch_types=[
            pltpu.VMEM((chunks_per_w, _C), jnp.int32),
            pltpu.VMEM((_C, _D), jnp.float32),
            pltpu.VMEM((_C, _D), jnp.float32),
            pltpu.SemaphoreType.DMA,
            pltpu.SemaphoreType.DMA,
        ],
    )
    def gather_kernel(idx_hbm, table_hbm, out_hbm, idx_v, buf0, buf1,
                      sem0, sem1):
        wid = lax.axis_index("s") * 2 + lax.axis_index("c")
        chunk0 = wid * chunks_per_w
        pltpu.sync_copy(idx_hbm.at[pl.ds(chunk0, chunks_per_w)], idx_v)

        def body(p, carry):
            c = 2 * p
            g0 = pltpu.async_copy(table_hbm.at[idx_v.at[c]], buf0, sem0)
            g1 = pltpu.async_copy(table_hbm.at[idx_v.at[c + 1]], buf1, sem1)
            row = (chunk0 + c) * _C
            g0.wait()
            pltpu.sync_copy(buf0, out_hbm.at[pl.ds(row, _C)])
            g1.wait()
            pltpu.sync_copy(buf1, out_hbm.at[pl.ds(row + _C, _C)])
            return carry

        lax.fori_loop(0, chunks_per_w // 2, body, 0)

    return gather_kernel


def kernel(sentence, word_vectors):
    batch, seq = sentence.shape
    n_tokens = batch * seq
    idx = sentence.reshape(n_tokens // _C, _C).astype(jnp.int32)
    out = _make_gather(n_tokens, word_vectors.shape[0])(idx, word_vectors)
    return out.reshape(batch, seq, _D)


# speedup vs baseline: 6.9835x; 1.0707x over previous
import functools

import jax
import jax.numpy as jnp
from jax import lax
from jax.experimental import pallas as pl
from jax.experimental.pallas import tpu as pltpu
from jax.experimental.pallas import tpu_sc as plsc

_D = 64
_C = 128
_NW = 32
_RING = 8


@functools.lru_cache(maxsize=None)
def _make_gather(n_tokens: int):
    assert n_tokens % (_NW * _C * _RING) == 0
    chunks_per_w = n_tokens // (_NW * _C)
    n_rounds = chunks_per_w // _RING
    mesh = plsc.VectorSubcoreMesh(core_axis_name="c", subcore_axis_name="s")

    scratch = (
        [pltpu.VMEM((chunks_per_w, _C), jnp.int32)]
        + [pltpu.VMEM((_C, _D), jnp.float32) for _ in range(_RING)]
        + [pltpu.SemaphoreType.DMA for _ in range(2 * _RING)]
    )

    @functools.partial(
        pl.kernel,
        mesh=mesh,
        compiler_params=pltpu.CompilerParams(use_tc_tiling_on_sc=False),
        out_type=jax.ShapeDtypeStruct((n_tokens, _D), jnp.float32),
        scratch_types=scratch,
    )
    def gather_kernel(idx_hbm, table_hbm, out_hbm, idx_v, *rest):
        bufs = rest[:_RING]
        gsem = rest[_RING:2 * _RING]
        wsem = rest[2 * _RING:]
        wid = lax.axis_index("s") * 2 + lax.axis_index("c")
        chunk0 = wid * chunks_per_w
        pltpu.sync_copy(idx_hbm.at[pl.ds(chunk0, chunks_per_w)], idx_v)

        def round_body(p, carry):
            c = _RING * p
            for j in range(_RING):
                @pl.when(p > 0)
                def _(j=j, c=c):
                    row = (chunk0 + c - _RING + j) * _C
                    pltpu.make_async_copy(
                        bufs[j], out_hbm.at[pl.ds(row, _C)], wsem[j]).wait()
                pltpu.async_copy(table_hbm.at[idx_v.at[c + j]], bufs[j],
                                 gsem[j])
            for j in range(_RING):
                pltpu.make_async_copy(table_hbm.at[idx_v.at[c + j]], bufs[j],
                                      gsem[j]).wait()
                row = (chunk0 + c + j) * _C
                pltpu.async_copy(bufs[j], out_hbm.at[pl.ds(row, _C)], wsem[j])
            return carry

        lax.fori_loop(0, n_rounds, round_body, 0)
        for j in range(_RING):
            row = (chunk0 + chunks_per_w - _RING + j) * _C
            pltpu.make_async_copy(
                bufs[j], out_hbm.at[pl.ds(row, _C)], wsem[j]).wait()

    return gather_kernel


def kernel(sentence, word_vectors):
    batch, seq = sentence.shape
    n_tokens = batch * seq
    idx = sentence.reshape(n_tokens // _C, _C).astype(jnp.int32)
    out = _make_gather(n_tokens)(idx, word_vectors)
    return out.reshape(batch, seq, _D)
